# baseline (speedup 1.0000x reference)
import functools

import jax
import jax.numpy as jnp
from jax import lax
from jax.experimental import pallas as pl
from jax.experimental.pallas import tpu as pltpu
from jax.experimental.pallas import tpu_sc as plsc

B, S, H = 1, 2048, 1024
I = 512
E = 16
K = 4
G = 4
EPG = E // G
TG = 2
SCALE = 2.5
T = B * S
TM = 128
NP = (T * K) // TM + E
P = NP * TM
H2 = H // 2
NEG = -1e30


def _gate_body(x_ref, gwt_ref, bias_ref, pos_ref, w_ref, eid_ref, xbf_ref):
    x = x_ref[...]
    xbf_ref[...] = x.astype(jnp.bfloat16)
    logits = jnp.dot(x, gwt_ref[...], preferred_element_type=jnp.float32)
    scores = 1.0 / (1.0 + jnp.exp(-logits))
    sfc = scores + bias_ref[...]
    lane = jax.lax.broadcasted_iota(jnp.int32, (T, E), 1)

    gs = jnp.zeros((T, G), jnp.float32)
    lane4 = jax.lax.broadcasted_iota(jnp.int32, (T, G), 1)
    for g in range(G):
        m = (lane // EPG) == g
        vals = jnp.where(m, sfc, NEG)
        m1 = jnp.max(vals, axis=-1, keepdims=True)
        idx1 = jnp.min(jnp.where((vals == m1) & m, lane, E), axis=-1,
                       keepdims=True)
        m2 = jnp.max(jnp.where(lane == idx1, NEG, vals), axis=-1,
                     keepdims=True)
        gs = gs + jnp.where(lane4 == g, m1 + m2, 0.0)

    g1v = jnp.max(gs, axis=-1, keepdims=True)
    g1 = jnp.min(jnp.where(gs == g1v, lane4, G), axis=-1, keepdims=True)
    gs2 = jnp.where(lane4 == g1, NEG, gs)
    g2v = jnp.max(gs2, axis=-1, keepdims=True)
    g2 = jnp.min(jnp.where(gs2 == g2v, lane4, G), axis=-1, keepdims=True)
    grp = lane // EPG
    smask = (grp == g1) | (grp == g2)
    tmp = jnp.where(smask, sfc, 0.0)

    oh_k = []
    w_cols = jnp.zeros((T, E), jnp.float32)
    cur = tmp
    for k in range(K):
        mk = jnp.max(cur, axis=-1, keepdims=True)
        ik = jnp.min(jnp.where(cur == mk, lane, E), axis=-1, keepdims=True)
        sel = (lane == ik)
        wk = jnp.sum(jnp.where(sel, scores, 0.0), axis=-1, keepdims=True)
        oh_k.append(sel.astype(jnp.float32))
        w_cols = w_cols + jnp.where(lane == k, wk, 0.0)
        cur = jnp.where(sel, NEG, cur)
    wsum = jnp.sum(jnp.where(lane < K, w_cols, 0.0), axis=-1, keepdims=True)
    w_ref[...] = w_cols / (wsum + 1e-20) * SCALE

    OH = oh_k[0] + oh_k[1] + oh_k[2] + oh_k[3]
    ii = jax.lax.broadcasted_iota(jnp.int32, (T, T), 0)
    jj = jax.lax.broadcasted_iota(jnp.int32, (T, T), 1)
    Lstrict = (jj < ii).astype(jnp.float32)
    CUM = jnp.dot(Lstrict, OH, preferred_element_type=jnp.float32)
    counts_row = jnp.sum(OH, axis=0, keepdims=True)
    cnt_pad_row = jnp.ceil(counts_row / TM) * TM
    ei = jax.lax.broadcasted_iota(jnp.int32, (E, E), 0)
    ej = jax.lax.broadcasted_iota(jnp.int32, (E, E), 1)
    offs_row = jnp.sum(jnp.where(ei < ej, cnt_pad_row.T, 0.0), axis=0,
                       keepdims=True)
    base = offs_row + CUM
    for k in range(K):
        p = jnp.sum(oh_k[k] * base, axis=-1, keepdims=True)
        pos_ref[:, k:k + 1] = p.astype(jnp.int32)

    offs_col = jnp.sum(jnp.where(ej < ei, cnt_pad_row, 0.0), axis=1,
                       keepdims=True)
    cnt_pad_col = jnp.sum(jnp.where(ei == ej, cnt_pad_row, 0.0), axis=1,
                          keepdims=True)
    ends_col = offs_col + cnt_pad_col
    ti = jax.lax.broadcasted_iota(jnp.int32, (E, 128), 1).astype(jnp.float32)
    cmp = (ti * TM >= ends_col).astype(jnp.int32)
    eid = jnp.minimum(jnp.sum(cmp, axis=0, keepdims=True), E - 1)
    eid_ref[...] = eid.astype(jnp.int32)


def _gate_call(x2, gwt, bias2):
    return pl.pallas_call(
        _gate_body,
        out_shape=[
            jax.ShapeDtypeStruct((T, K), jnp.int32),
            jax.ShapeDtypeStruct((T, E), jnp.float32),
            jax.ShapeDtypeStruct((1, 128), jnp.int32),
            jax.ShapeDtypeStruct((T, H), jnp.bfloat16),
        ],
    )(x2, gwt, bias2)


def _silu(x):
    return x / (1.0 + jnp.exp(-x))


def _ffn_body(eid_ref, x_ref, gp_ref, up_ref, dp_ref, o_ref):
    x = x_ref[...]
    h1 = jax.lax.dot_general(x, gp_ref[0], (((1,), (1,)), ((), ())),
                             preferred_element_type=jnp.float32)
    h2 = jax.lax.dot_general(x, up_ref[0], (((1,), (1,)), ((), ())),
                             preferred_element_type=jnp.float32)
    act = (_silu(h1) * h2).astype(jnp.bfloat16)
    y = jax.lax.dot_general(act, dp_ref[0], (((1,), (1,)), ((), ())),
                            preferred_element_type=jnp.float32)
    o_ref[...] = y.astype(jnp.bfloat16)


def _ffn_call(eid, xs, gp, up, dp):
    grid_spec = pltpu.PrefetchScalarGridSpec(
        num_scalar_prefetch=1,
        grid=(NP,),
        in_specs=[
            pl.BlockSpec((TM, H), lambda i, eid: (i, 0)),
            pl.BlockSpec((1, I, H), lambda i, eid: (eid[i], 0, 0)),
            pl.BlockSpec((1, I, H), lambda i, eid: (eid[i], 0, 0)),
            pl.BlockSpec((1, H, I), lambda i, eid: (eid[i], 0, 0)),
        ],
        out_specs=pl.BlockSpec((TM, H), lambda i, eid: (i, 0)),
    )
    return pl.pallas_call(
        _ffn_body,
        grid_spec=grid_spec,
        out_shape=jax.ShapeDtypeStruct((P, H), jnp.bfloat16),
    )(eid, xs, gp, up, dp)


def _shared_body(x_ref, gw_ref, uw_ref, dw_ref, ysg_ref, w_ref, o_ref):
    x = x_ref[...]
    sg = jax.lax.dot_general(x, gw_ref[...], (((1,), (1,)), ((), ())),
                             preferred_element_type=jnp.float32)
    su = jax.lax.dot_general(x, uw_ref[...], (((1,), (1,)), ((), ())),
                             preferred_element_type=jnp.float32)
    act = (_silu(sg) * su).astype(jnp.bfloat16)
    out = jax.lax.dot_general(act, dw_ref[...], (((1,), (1,)), ((), ())),
                              preferred_element_type=jnp.float32)
    for k in range(K):
        out = out + w_ref[:, k:k + 1] * ysg_ref[k].astype(jnp.float32)
    o_ref[...] = out


def _shared_call(x2, gw, uw, dw, ysg, w16):
    TS = 256
    return pl.pallas_call(
        _shared_body,
        grid=(T // TS,),
        in_specs=[
            pl.BlockSpec((TS, H), lambda i: (i, 0)),
            pl.BlockSpec((I, H), lambda i: (0, 0)),
            pl.BlockSpec((I, H), lambda i: (0, 0)),
            pl.BlockSpec((H, I), lambda i: (0, 0)),
            pl.BlockSpec((K, TS, H), lambda i: (0, i, 0)),
            pl.BlockSpec((TS, E), lambda i: (i, 0)),
        ],
        out_specs=pl.BlockSpec((TS, H), lambda i: (i, 0)),
        out_shape=jax.ShapeDtypeStruct((T, H), jnp.float32),
    )(x2, gw, uw, dw, ysg, w16)


_SC_MESH = plsc.VectorSubcoreMesh(core_axis_name="c", subcore_axis_name="s")
_CH = 32
_NCH = 256 // _CH


def _dispatch_body(x2_hbm, post_hbm, xs_hbm, xb0, xb1, ib0, ib1, sem0, sem1):
    wid = lax.axis_index("s") * 2 + lax.axis_index("c")
    k = wid // 8
    stripe = wid % 8
    xbufs, ibufs, sems = (xb0, xb1), (ib0, ib1), (sem0, sem1)
    pend = [None, None]
    for c in range(_NCH):
        tb = stripe * 256 + c * _CH
        xb, ib, sm = xbufs[c % 2], ibufs[c % 2], sems[c % 2]
        if pend[c % 2] is not None:
            pend[c % 2].wait()
        pltpu.sync_copy(post_hbm.at[k, pl.ds(tb, _CH)], ib)
        pltpu.sync_copy(x2_hbm.at[pl.ds(tb, _CH)], xb)
        pend[c % 2] = pltpu.async_copy(xb, xs_hbm.at[ib], sm)
    for d in pend:
        if d is not None:
            d.wait()


@functools.partial(
    pl.kernel,
    out_type=jax.ShapeDtypeStruct((P, H2), jnp.int32),
    mesh=_SC_MESH,
    scratch_types=[
        pltpu.VMEM((_CH, H2), jnp.int32),
        pltpu.VMEM((_CH, H2), jnp.int32),
        pltpu.VMEM((_CH,), jnp.int32),
        pltpu.VMEM((_CH,), jnp.int32),
        pltpu.SemaphoreType.DMA,
        pltpu.SemaphoreType.DMA,
    ],
)
def _dispatch_sc(x2, post, xs, xb0, xb1, ib0, ib1, sem0, sem1):
    _dispatch_body(x2, post, xs, xb0, xb1, ib0, ib1, sem0, sem1)


def _gather_body(ys_hbm, post_hbm, ysg_hbm, gb0, gb1, ib0, ib1, sem0, sem1):
    wid = lax.axis_index("s") * 2 + lax.axis_index("c")
    k = wid // 8
    stripe = wid % 8
    gbufs, ibufs, sems = (gb0, gb1), (ib0, ib1), (sem0, sem1)
    pend = [None, None]
    for c in range(_NCH):
        tb = stripe * 256 + c * _CH
        gb, ib, sm = gbufs[c % 2], ibufs[c % 2], sems[c % 2]
        if pend[c % 2] is not None:
            pend[c % 2].wait()
        pltpu.sync_copy(post_hbm.at[k, pl.ds(tb, _CH)], ib)
        pltpu.async_copy(ys_hbm.at[ib], gb, sm).wait()
        pend[c % 2] = pltpu.async_copy(gb, ysg_hbm.at[k, pl.ds(tb, _CH)], sm)
    for d in pend:
        if d is not None:
            d.wait()


@functools.partial(
    pl.kernel,
    out_type=jax.ShapeDtypeStruct((K, T, H2), jnp.int32),
    mesh=_SC_MESH,
    scratch_types=[
        pltpu.VMEM((_CH, H2), jnp.int32),
        pltpu.VMEM((_CH, H2), jnp.int32),
        pltpu.VMEM((_CH,), jnp.int32),
        pltpu.VMEM((_CH,), jnp.int32),
        pltpu.SemaphoreType.DMA,
        pltpu.SemaphoreType.DMA,
    ],
)
def _gather_sc(ys, post, ysg, gb0, gb1, ib0, ib1, sem0, sem1):
    _gather_body(ys, post, ysg, gb0, gb1, ib0, ib1, sem0, sem1)


def kernel(hidden_states, gate_weight, e_score_correction_bias, gate_proj,
           up_proj, down_proj, shared_gate_w, shared_up_w, shared_down_w):
    x2 = hidden_states.reshape(T, H)
    pos, w16, eid_pad, x2bf = _gate_call(x2, gate_weight.T,
                                         e_score_correction_bias.reshape(1, E))
    eid = eid_pad[0, :NP]
    post = pos.T

    bf = jnp.bfloat16

    def _to_i32(a):
        return lax.bitcast_convert_type(a.reshape(*a.shape[:-1], H2, 2),
                                        jnp.int32)

    def _to_bf16(a):
        return lax.bitcast_convert_type(a, bf).reshape(*a.shape[:-1], H)

    xs = _to_bf16(_dispatch_sc(_to_i32(x2bf), post))
    ys = _ffn_call(eid, xs, gate_proj.astype(bf), up_proj.astype(bf),
                   down_proj.astype(bf))
    ysg = _to_bf16(_gather_sc(_to_i32(ys), post))
    out = _shared_call(x2bf, shared_gate_w.astype(bf), shared_up_w.astype(bf),
                       shared_down_w.astype(bf), ysg, w16)
    return out.reshape(B, S, H)


# speedup vs baseline: 2.9496x; 2.9496x over previous
import functools

import jax
import jax.numpy as jnp
from jax import lax
from jax.experimental import pallas as pl
from jax.experimental.pallas import tpu as pltpu
from jax.experimental.pallas import tpu_sc as plsc

B, S, H = 1, 2048, 1024
I = 512
E = 16
K = 4
G = 4
EPG = E // G
TG = 2
SCALE = 2.5
T = B * S
TM = 128
NP = (T * K) // TM + E
P = NP * TM
H2 = H // 2
NEG = -1e30


def _gate_body(x_ref, gwt_ref, bias_ref, pos_ref, w_ref, eid_ref):
    x = x_ref[...]
    logits = jnp.dot(x, gwt_ref[...], preferred_element_type=jnp.float32)
    scores = 1.0 / (1.0 + jnp.exp(-logits))
    sfc = scores + bias_ref[...]
    lane = jax.lax.broadcasted_iota(jnp.int32, (T, E), 1)

    gs = jnp.zeros((T, G), jnp.float32)
    lane4 = jax.lax.broadcasted_iota(jnp.int32, (T, G), 1)
    for g in range(G):
        m = (lane // EPG) == g
        vals = jnp.where(m, sfc, NEG)
        m1 = jnp.max(vals, axis=-1, keepdims=True)
        idx1 = jnp.min(jnp.where((vals == m1) & m, lane, E), axis=-1,
                       keepdims=True)
        m2 = jnp.max(jnp.where(lane == idx1, NEG, vals), axis=-1,
                     keepdims=True)
        gs = gs + jnp.where(lane4 == g, m1 + m2, 0.0)

    g1v = jnp.max(gs, axis=-1, keepdims=True)
    g1 = jnp.min(jnp.where(gs == g1v, lane4, G), axis=-1, keepdims=True)
    gs2 = jnp.where(lane4 == g1, NEG, gs)
    g2v = jnp.max(gs2, axis=-1, keepdims=True)
    g2 = jnp.min(jnp.where(gs2 == g2v, lane4, G), axis=-1, keepdims=True)
    grp = lane // EPG
    smask = (grp == g1) | (grp == g2)
    tmp = jnp.where(smask, sfc, 0.0)

    oh_k = []
    w_cols = jnp.zeros((T, E), jnp.float32)
    cur = tmp
    for k in range(K):
        mk = jnp.max(cur, axis=-1, keepdims=True)
        ik = jnp.min(jnp.where(cur == mk, lane, E), axis=-1, keepdims=True)
        sel = (lane == ik)
        wk = jnp.sum(jnp.where(sel, scores, 0.0), axis=-1, keepdims=True)
        oh_k.append(sel.astype(jnp.float32))
        w_cols = w_cols + jnp.where(lane == k, wk, 0.0)
        cur = jnp.where(sel, NEG, cur)
    wsum = jnp.sum(jnp.where(lane < K, w_cols, 0.0), axis=-1, keepdims=True)
    w_ref[...] = w_cols / (wsum + 1e-20) * SCALE

    OH = oh_k[0] + oh_k[1] + oh_k[2] + oh_k[3]
    ii = jax.lax.broadcasted_iota(jnp.int32, (T, T), 0)
    jj = jax.lax.broadcasted_iota(jnp.int32, (T, T), 1)
    Lstrict = (jj < ii).astype(jnp.float32)
    CUM = jnp.dot(Lstrict, OH, preferred_element_type=jnp.float32)
    counts_row = jnp.sum(OH, axis=0, keepdims=True)
    cnt_pad_row = jnp.ceil(counts_row / TM) * TM
    ei = jax.lax.broadcasted_iota(jnp.int32, (E, E), 0)
    ej = jax.lax.broadcasted_iota(jnp.int32, (E, E), 1)
    offs_row = jnp.sum(jnp.where(ei < ej, cnt_pad_row.T, 0.0), axis=0,
                       keepdims=True)
    base = offs_row + CUM
    for k in range(K):
        p = jnp.sum(oh_k[k] * base, axis=-1, keepdims=True)
        pos_ref[:, k:k + 1] = p.astype(jnp.int32)

    offs_col = jnp.sum(jnp.where(ej < ei, cnt_pad_row, 0.0), axis=1,
                       keepdims=True)
    cnt_pad_col = jnp.sum(jnp.where(ei == ej, cnt_pad_row, 0.0), axis=1,
                          keepdims=True)
    ends_col = offs_col + cnt_pad_col
    ti = jax.lax.broadcasted_iota(jnp.int32, (E, 128), 1).astype(jnp.float32)
    cmp = (ti * TM >= ends_col).astype(jnp.int32)
    eid = jnp.minimum(jnp.sum(cmp, axis=0, keepdims=True), E - 1)
    eid_ref[...] = eid.astype(jnp.int32)


def _gate_call(x2, gwt, bias2):
    return pl.pallas_call(
        _gate_body,
        out_shape=[
            jax.ShapeDtypeStruct((T, K), jnp.int32),
            jax.ShapeDtypeStruct((T, E), jnp.float32),
            jax.ShapeDtypeStruct((1, 128), jnp.int32),
        ],
    )(x2, gwt, bias2)


def _silu(x):
    return x / (1.0 + jnp.exp(-x))


def _ffn_body(eid_ref, x_ref, gp_ref, up_ref, dp_ref, o_ref):
    x = x_ref[...].astype(jnp.bfloat16)
    h1 = jax.lax.dot_general(x, gp_ref[0], (((1,), (1,)), ((), ())),
                             preferred_element_type=jnp.float32)
    h2 = jax.lax.dot_general(x, up_ref[0], (((1,), (1,)), ((), ())),
                             preferred_element_type=jnp.float32)
    act = (_silu(h1) * h2).astype(jnp.bfloat16)
    o_ref[...] = jax.lax.dot_general(act, dp_ref[0], (((1,), (1,)), ((), ())),
                                     preferred_element_type=jnp.float32)


def _ffn_call(eid, xs, gp, up, dp):
    grid_spec = pltpu.PrefetchScalarGridSpec(
        num_scalar_prefetch=1,
        grid=(NP,),
        in_specs=[
            pl.BlockSpec((TM, H), lambda i, eid: (i, 0)),
            pl.BlockSpec((1, I, H), lambda i, eid: (eid[i], 0, 0)),
            pl.BlockSpec((1, I, H), lambda i, eid: (eid[i], 0, 0)),
            pl.BlockSpec((1, H, I), lambda i, eid: (eid[i], 0, 0)),
        ],
        out_specs=pl.BlockSpec((TM, H), lambda i, eid: (i, 0)),
    )
    return pl.pallas_call(
        _ffn_body,
        grid_spec=grid_spec,
        out_shape=jax.ShapeDtypeStruct((P, H), jnp.float32),
    )(eid, xs, gp, up, dp)


def _shared_body(x_ref, gw_ref, uw_ref, dw_ref, ysg_ref, w_ref, o_ref):
    x = x_ref[...].astype(jnp.bfloat16)
    sg = jax.lax.dot_general(x, gw_ref[...], (((1,), (1,)), ((), ())),
                             preferred_element_type=jnp.float32)
    su = jax.lax.dot_general(x, uw_ref[...], (((1,), (1,)), ((), ())),
                             preferred_element_type=jnp.float32)
    act = (_silu(sg) * su).astype(jnp.bfloat16)
    out = jax.lax.dot_general(act, dw_ref[...], (((1,), (1,)), ((), ())),
                              preferred_element_type=jnp.float32)
    for k in range(K):
        out = out + w_ref[:, k:k + 1] * ysg_ref[k]
    o_ref[...] = out


def _shared_call(x2, gw, uw, dw, ysg, w16):
    TS = 256
    return pl.pallas_call(
        _shared_body,
        grid=(T // TS,),
        in_specs=[
            pl.BlockSpec((TS, H), lambda i: (i, 0)),
            pl.BlockSpec((I, H), lambda i: (0, 0)),
            pl.BlockSpec((I, H), lambda i: (0, 0)),
            pl.BlockSpec((H, I), lambda i: (0, 0)),
            pl.BlockSpec((K, TS, H), lambda i: (0, i, 0)),
            pl.BlockSpec((TS, E), lambda i: (i, 0)),
        ],
        out_specs=pl.BlockSpec((TS, H), lambda i: (i, 0)),
        out_shape=jax.ShapeDtypeStruct((T, H), jnp.float32),
    )(x2, gw, uw, dw, ysg, w16)


_SC_MESH = plsc.VectorSubcoreMesh(core_axis_name="c", subcore_axis_name="s")
_CH = 32
_NCH = 256 // _CH


def _dispatch_body(x2_hbm, post_hbm, xs_hbm, xb0, xb1, ib0, ib1, sem0, sem1):
    wid = lax.axis_index("s") * 2 + lax.axis_index("c")
    k = wid // 8
    stripe = wid % 8
    xbufs, ibufs, sems = (xb0, xb1), (ib0, ib1), (sem0, sem1)
    pend = [None, None]
    for c in range(_NCH):
        tb = stripe * 256 + c * _CH
        xb, ib, sm = xbufs[c % 2], ibufs[c % 2], sems[c % 2]
        if pend[c % 2] is not None:
            pend[c % 2].wait()
        pltpu.sync_copy(post_hbm.at[k, pl.ds(tb, _CH)], ib)
        pltpu.sync_copy(x2_hbm.at[pl.ds(tb, _CH)], xb)
        pend[c % 2] = pltpu.async_copy(xb, xs_hbm.at[ib], sm)
    for d in pend:
        if d is not None:
            d.wait()


@functools.partial(
    pl.kernel,
    out_type=jax.ShapeDtypeStruct((P, H), jnp.float32),
    mesh=_SC_MESH,
    scratch_types=[
        pltpu.VMEM((_CH, H), jnp.float32),
        pltpu.VMEM((_CH, H), jnp.float32),
        pltpu.VMEM((_CH,), jnp.int32),
        pltpu.VMEM((_CH,), jnp.int32),
        pltpu.SemaphoreType.DMA,
        pltpu.SemaphoreType.DMA,
    ],
)
def _dispatch_sc(x2, post, xs, xb0, xb1, ib0, ib1, sem0, sem1):
    _dispatch_body(x2, post, xs, xb0, xb1, ib0, ib1, sem0, sem1)


def _gather_body(ys_hbm, post_hbm, ysg_hbm, gb0, gb1, ib0, ib1, sem0, sem1):
    wid = lax.axis_index("s") * 2 + lax.axis_index("c")
    k = wid // 8
    stripe = wid % 8
    gbufs, ibufs, sems = (gb0, gb1), (ib0, ib1), (sem0, sem1)
    pend = [None, None]
    for c in range(_NCH):
        tb = stripe * 256 + c * _CH
        gb, ib, sm = gbufs[c % 2], ibufs[c % 2], sems[c % 2]
        if pend[c % 2] is not None:
            pend[c % 2].wait()
        pltpu.sync_copy(post_hbm.at[k, pl.ds(tb, _CH)], ib)
        pltpu.async_copy(ys_hbm.at[ib], gb, sm).wait()
        pend[c % 2] = pltpu.async_copy(gb, ysg_hbm.at[k, pl.ds(tb, _CH)], sm)
    for d in pend:
        if d is not None:
            d.wait()


@functools.partial(
    pl.kernel,
    out_type=jax.ShapeDtypeStruct((K, T, H), jnp.float32),
    mesh=_SC_MESH,
    scratch_types=[
        pltpu.VMEM((_CH, H), jnp.float32),
        pltpu.VMEM((_CH, H), jnp.float32),
        pltpu.VMEM((_CH,), jnp.int32),
        pltpu.VMEM((_CH,), jnp.int32),
        pltpu.SemaphoreType.DMA,
        pltpu.SemaphoreType.DMA,
    ],
)
def _gather_sc(ys, post, ysg, gb0, gb1, ib0, ib1, sem0, sem1):
    _gather_body(ys, post, ysg, gb0, gb1, ib0, ib1, sem0, sem1)


def kernel(hidden_states, gate_weight, e_score_correction_bias, gate_proj,
           up_proj, down_proj, shared_gate_w, shared_up_w, shared_down_w):
    x2 = hidden_states.reshape(T, H)
    pos, w16, eid_pad = _gate_call(x2, gate_weight.T,
                                   e_score_correction_bias.reshape(1, E))
    eid = eid_pad[0, :NP]
    post = pos.T

    bf = jnp.bfloat16
    xs = _dispatch_sc(x2, post)
    ys = _ffn_call(eid, xs, gate_proj.astype(bf), up_proj.astype(bf),
                   down_proj.astype(bf))
    ysg = _gather_sc(ys, post)
    out = _shared_call(x2, shared_gate_w.astype(bf), shared_up_w.astype(bf),
                       shared_down_w.astype(bf), ysg, w16)
    return out.reshape(B, S, H)


# speedup vs baseline: 3.2276x; 1.0943x over previous
import functools

import jax
import jax.numpy as jnp
from jax import lax
from jax.experimental import pallas as pl
from jax.experimental.pallas import tpu as pltpu
from jax.experimental.pallas import tpu_sc as plsc

B, S, H = 1, 2048, 1024
I = 512
E = 16
K = 4
G = 4
EPG = E // G
TG = 2
SCALE = 2.5
T = B * S
TM = 128
NP = (T * K) // TM + E
P = NP * TM
H2 = H // 2
NEG = -1e30


def _gate_body(x_ref, gwt_ref, bias_ref, pos_ref, w_ref, eid_ref):
    x = x_ref[...]
    logits = jnp.dot(x, gwt_ref[...], preferred_element_type=jnp.float32)
    scores = 1.0 / (1.0 + jnp.exp(-logits))
    sfc = scores + bias_ref[...]
    lane = jax.lax.broadcasted_iota(jnp.int32, (T, E), 1)

    gs = jnp.zeros((T, G), jnp.float32)
    lane4 = jax.lax.broadcasted_iota(jnp.int32, (T, G), 1)
    for g in range(G):
        m = (lane // EPG) == g
        vals = jnp.where(m, sfc, NEG)
        m1 = jnp.max(vals, axis=-1, keepdims=True)
        idx1 = jnp.min(jnp.where((vals == m1) & m, lane, E), axis=-1,
                       keepdims=True)
        m2 = jnp.max(jnp.where(lane == idx1, NEG, vals), axis=-1,
                     keepdims=True)
        gs = gs + jnp.where(lane4 == g, m1 + m2, 0.0)

    g1v = jnp.max(gs, axis=-1, keepdims=True)
    g1 = jnp.min(jnp.where(gs == g1v, lane4, G), axis=-1, keepdims=True)
    gs2 = jnp.where(lane4 == g1, NEG, gs)
    g2v = jnp.max(gs2, axis=-1, keepdims=True)
    g2 = jnp.min(jnp.where(gs2 == g2v, lane4, G), axis=-1, keepdims=True)
    grp = lane // EPG
    smask = (grp == g1) | (grp == g2)
    tmp = jnp.where(smask, sfc, 0.0)

    oh_k = []
    w_cols = jnp.zeros((T, E), jnp.float32)
    cur = tmp
    for k in range(K):
        mk = jnp.max(cur, axis=-1, keepdims=True)
        ik = jnp.min(jnp.where(cur == mk, lane, E), axis=-1, keepdims=True)
        sel = (lane == ik)
        wk = jnp.sum(jnp.where(sel, scores, 0.0), axis=-1, keepdims=True)
        oh_k.append(sel.astype(jnp.float32))
        w_cols = w_cols + jnp.where(lane == k, wk, 0.0)
        cur = jnp.where(sel, NEG, cur)
    wsum = jnp.sum(jnp.where(lane < K, w_cols, 0.0), axis=-1, keepdims=True)
    w_ref[...] = w_cols / (wsum + 1e-20) * SCALE

    OH = oh_k[0] + oh_k[1] + oh_k[2] + oh_k[3]
    ii = jax.lax.broadcasted_iota(jnp.int32, (T, T), 0)
    jj = jax.lax.broadcasted_iota(jnp.int32, (T, T), 1)
    Lstrict = (jj < ii).astype(jnp.float32)
    CUM = jnp.dot(Lstrict, OH, preferred_element_type=jnp.float32)
    counts_row = jnp.sum(OH, axis=0, keepdims=True)
    cnt_pad_row = jnp.ceil(counts_row / TM) * TM
    ei = jax.lax.broadcasted_iota(jnp.int32, (E, E), 0)
    ej = jax.lax.broadcasted_iota(jnp.int32, (E, E), 1)
    offs_row = jnp.sum(jnp.where(ei < ej, cnt_pad_row.T, 0.0), axis=0,
                       keepdims=True)
    base = offs_row + CUM
    for k in range(K):
        p = jnp.sum(oh_k[k] * base, axis=-1, keepdims=True)
        pos_ref[:, k:k + 1] = p.astype(jnp.int32)

    offs_col = jnp.sum(jnp.where(ej < ei, cnt_pad_row, 0.0), axis=1,
                       keepdims=True)
    cnt_pad_col = jnp.sum(jnp.where(ei == ej, cnt_pad_row, 0.0), axis=1,
                          keepdims=True)
    ends_col = offs_col + cnt_pad_col
    ti = jax.lax.broadcasted_iota(jnp.int32, (E, 128), 1).astype(jnp.float32)
    cmp = (ti * TM >= ends_col).astype(jnp.int32)
    eid = jnp.minimum(jnp.sum(cmp, axis=0, keepdims=True), E - 1)
    eid_ref[...] = eid.astype(jnp.int32)


def _gate_call(x2, gwt, bias2):
    return pl.pallas_call(
        _gate_body,
        out_shape=[
            jax.ShapeDtypeStruct((T, K), jnp.int32),
            jax.ShapeDtypeStruct((T, E), jnp.float32),
            jax.ShapeDtypeStruct((1, 128), jnp.int32),
        ],
    )(x2, gwt, bias2)


def _silu(x):
    return x / (1.0 + jnp.exp(-x))


def _ffn_body(eid_ref, x_ref, gp_ref, up_ref, dp_ref, o_ref):
    x = x_ref[...]
    h1 = jax.lax.dot_general(x, gp_ref[0], (((1,), (1,)), ((), ())),
                             preferred_element_type=jnp.float32)
    h2 = jax.lax.dot_general(x, up_ref[0], (((1,), (1,)), ((), ())),
                             preferred_element_type=jnp.float32)
    act = _silu(h1) * h2
    o_ref[...] = jax.lax.dot_general(act, dp_ref[0], (((1,), (1,)), ((), ())),
                                     preferred_element_type=jnp.float32)


def _ffn_call(eid, xs, gp, up, dp):
    grid_spec = pltpu.PrefetchScalarGridSpec(
        num_scalar_prefetch=1,
        grid=(NP,),
        in_specs=[
            pl.BlockSpec((TM, H), lambda i, eid: (i, 0)),
            pl.BlockSpec((1, I, H), lambda i, eid: (eid[i], 0, 0)),
            pl.BlockSpec((1, I, H), lambda i, eid: (eid[i], 0, 0)),
            pl.BlockSpec((1, H, I), lambda i, eid: (eid[i], 0, 0)),
        ],
        out_specs=pl.BlockSpec((TM, H), lambda i, eid: (i, 0)),
    )
    return pl.pallas_call(
        _ffn_body,
        grid_spec=grid_spec,
        out_shape=jax.ShapeDtypeStruct((P, H), jnp.float32),
    )(eid, xs, gp, up, dp)


def _shared_body(x_ref, gw_ref, uw_ref, dw_ref, ysg_ref, w_ref, o_ref):
    x = x_ref[...]
    sg = jax.lax.dot_general(x, gw_ref[...], (((1,), (1,)), ((), ())),
                             preferred_element_type=jnp.float32)
    su = jax.lax.dot_general(x, uw_ref[...], (((1,), (1,)), ((), ())),
                             preferred_element_type=jnp.float32)
    act = _silu(sg) * su
    out = jax.lax.dot_general(act, dw_ref[...], (((1,), (1,)), ((), ())),
                              preferred_element_type=jnp.float32)
    for k in range(K):
        out = out + w_ref[:, k:k + 1] * ysg_ref[k]
    o_ref[...] = out


def _shared_call(x2, gw, uw, dw, ysg, w16):
    TS = 256
    return pl.pallas_call(
        _shared_body,
        grid=(T // TS,),
        in_specs=[
            pl.BlockSpec((TS, H), lambda i: (i, 0)),
            pl.BlockSpec((I, H), lambda i: (0, 0)),
            pl.BlockSpec((I, H), lambda i: (0, 0)),
            pl.BlockSpec((H, I), lambda i: (0, 0)),
            pl.BlockSpec((K, TS, H), lambda i: (0, i, 0)),
            pl.BlockSpec((TS, E), lambda i: (i, 0)),
        ],
        out_specs=pl.BlockSpec((TS, H), lambda i: (i, 0)),
        out_shape=jax.ShapeDtypeStruct((T, H), jnp.float32),
    )(x2, gw, uw, dw, ysg, w16)


_SC_MESH = plsc.VectorSubcoreMesh(core_axis_name="c", subcore_axis_name="s")
_CH = 32
_NCH = 256 // _CH


def _dispatch_body(x2_hbm, post_hbm, xs_hbm, xb0, xb1, ib0, ib1, sem0, sem1):
    wid = lax.axis_index("s") * 2 + lax.axis_index("c")
    k = wid // 8
    stripe = wid % 8
    xbufs, ibufs, sems = (xb0, xb1), (ib0, ib1), (sem0, sem1)
    pend = [None, None]
    for c in range(_NCH):
        tb = stripe * 256 + c * _CH
        xb, ib, sm = xbufs[c % 2], ibufs[c % 2], sems[c % 2]
        if pend[c % 2] is not None:
            pend[c % 2].wait()
        pltpu.sync_copy(post_hbm.at[k, pl.ds(tb, _CH)], ib)
        pltpu.sync_copy(x2_hbm.at[pl.ds(tb, _CH)], xb)
        pend[c % 2] = pltpu.async_copy(xb, xs_hbm.at[ib], sm)
    for d in pend:
        if d is not None:
            d.wait()


@functools.partial(
    pl.kernel,
    out_type=jax.ShapeDtypeStruct((P, H), jnp.float32),
    mesh=_SC_MESH,
    scratch_types=[
        pltpu.VMEM((_CH, H), jnp.float32),
        pltpu.VMEM((_CH, H), jnp.float32),
        pltpu.VMEM((_CH,), jnp.int32),
        pltpu.VMEM((_CH,), jnp.int32),
        pltpu.SemaphoreType.DMA,
        pltpu.SemaphoreType.DMA,
    ],
)
def _dispatch_sc(x2, post, xs, xb0, xb1, ib0, ib1, sem0, sem1):
    _dispatch_body(x2, post, xs, xb0, xb1, ib0, ib1, sem0, sem1)


def _gather_body(ys_hbm, post_hbm, ysg_hbm, gb0, gb1, ib0, ib1, sem0, sem1):
    wid = lax.axis_index("s") * 2 + lax.axis_index("c")
    k = wid // 8
    stripe = wid % 8
    gbufs, ibufs, sems = (gb0, gb1), (ib0, ib1), (sem0, sem1)
    pend = [None, None]
    for c in range(_NCH):
        tb = stripe * 256 + c * _CH
        gb, ib, sm = gbufs[c % 2], ibufs[c % 2], sems[c % 2]
        if pend[c % 2] is not None:
            pend[c % 2].wait()
        pltpu.sync_copy(post_hbm.at[k, pl.ds(tb, _CH)], ib)
        pltpu.async_copy(ys_hbm.at[ib], gb, sm).wait()
        pend[c % 2] = pltpu.async_copy(gb, ysg_hbm.at[k, pl.ds(tb, _CH)], sm)
    for d in pend:
        if d is not None:
            d.wait()


@functools.partial(
    pl.kernel,
    out_type=jax.ShapeDtypeStruct((K, T, H), jnp.float32),
    mesh=_SC_MESH,
    scratch_types=[
        pltpu.VMEM((_CH, H), jnp.float32),
        pltpu.VMEM((_CH, H), jnp.float32),
        pltpu.VMEM((_CH,), jnp.int32),
        pltpu.VMEM((_CH,), jnp.int32),
        pltpu.SemaphoreType.DMA,
        pltpu.SemaphoreType.DMA,
    ],
)
def _gather_sc(ys, post, ysg, gb0, gb1, ib0, ib1, sem0, sem1):
    _gather_body(ys, post, ysg, gb0, gb1, ib0, ib1, sem0, sem1)


def kernel(hidden_states, gate_weight, e_score_correction_bias, gate_proj,
           up_proj, down_proj, shared_gate_w, shared_up_w, shared_down_w):
    x2 = hidden_states.reshape(T, H)
    pos, w16, eid_pad = _gate_call(x2, gate_weight.T,
                                   e_score_correction_bias.reshape(1, E))
    eid = eid_pad[0, :NP]
    post = pos.T

    xs = _dispatch_sc(x2, post)
    ys = _ffn_call(eid, xs, gate_proj, up_proj, down_proj)
    ysg = _gather_sc(ys, post)
    out = _shared_call(x2, shared_gate_w, shared_up_w, shared_down_w, ysg, w16)
    return out.reshape(B, S, H)


# speedup vs baseline: 3.8022x; 1.1780x over previous
import functools

import jax
import jax.numpy as jnp
from jax import lax
from jax.experimental import pallas as pl
from jax.experimental.pallas import tpu as pltpu
from jax.experimental.pallas import tpu_sc as plsc

B, S, H = 1, 2048, 1024
I = 512
E = 16
K = 4
G = 4
EPG = E // G
TG = 2
SCALE = 2.5
T = B * S
TM = 256
NP = (T * K) // TM + E
P = NP * TM
H2 = H // 2
NEG = -1e30


def _gate_body(x_ref, gwt_ref, bias_ref, pos_ref, w_ref, eid_ref):
    x = x_ref[...]
    logits = jnp.dot(x, gwt_ref[...], preferred_element_type=jnp.float32)
    scores = 1.0 / (1.0 + jnp.exp(-logits))
    sfc = scores + bias_ref[...]
    lane = jax.lax.broadcasted_iota(jnp.int32, (T, E), 1)

    gs = jnp.zeros((T, G), jnp.float32)
    lane4 = jax.lax.broadcasted_iota(jnp.int32, (T, G), 1)
    for g in range(G):
        m = (lane // EPG) == g
        vals = jnp.where(m, sfc, NEG)
        m1 = jnp.max(vals, axis=-1, keepdims=True)
        idx1 = jnp.min(jnp.where((vals == m1) & m, lane, E), axis=-1,
                       keepdims=True)
        m2 = jnp.max(jnp.where(lane == idx1, NEG, vals), axis=-1,
                     keepdims=True)
        gs = gs + jnp.where(lane4 == g, m1 + m2, 0.0)

    g1v = jnp.max(gs, axis=-1, keepdims=True)
    g1 = jnp.min(jnp.where(gs == g1v, lane4, G), axis=-1, keepdims=True)
    gs2 = jnp.where(lane4 == g1, NEG, gs)
    g2v = jnp.max(gs2, axis=-1, keepdims=True)
    g2 = jnp.min(jnp.where(gs2 == g2v, lane4, G), axis=-1, keepdims=True)
    grp = lane // EPG
    smask = (grp == g1) | (grp == g2)
    tmp = jnp.where(smask, sfc, 0.0)

    oh_k = []
    w_cols = jnp.zeros((T, E), jnp.float32)
    cur = tmp
    for k in range(K):
        mk = jnp.max(cur, axis=-1, keepdims=True)
        ik = jnp.min(jnp.where(cur == mk, lane, E), axis=-1, keepdims=True)
        sel = (lane == ik)
        wk = jnp.sum(jnp.where(sel, scores, 0.0), axis=-1, keepdims=True)
        oh_k.append(sel.astype(jnp.float32))
        w_cols = w_cols + jnp.where(lane == k, wk, 0.0)
        cur = jnp.where(sel, NEG, cur)
    wsum = jnp.sum(jnp.where(lane < K, w_cols, 0.0), axis=-1, keepdims=True)
    w_ref[...] = w_cols / (wsum + 1e-20) * SCALE

    OH = oh_k[0] + oh_k[1] + oh_k[2] + oh_k[3]
    ii = jax.lax.broadcasted_iota(jnp.int32, (T, T), 0)
    jj = jax.lax.broadcasted_iota(jnp.int32, (T, T), 1)
    Lstrict = (jj < ii).astype(jnp.float32)
    CUM = jnp.dot(Lstrict, OH, preferred_element_type=jnp.float32)
    counts_row = jnp.sum(OH, axis=0, keepdims=True)
    cnt_pad_row = jnp.ceil(counts_row / TM) * TM
    ei = jax.lax.broadcasted_iota(jnp.int32, (E, E), 0)
    ej = jax.lax.broadcasted_iota(jnp.int32, (E, E), 1)
    offs_row = jnp.sum(jnp.where(ei < ej, cnt_pad_row.T, 0.0), axis=0,
                       keepdims=True)
    base = offs_row + CUM
    for k in range(K):
        p = jnp.sum(oh_k[k] * base, axis=-1, keepdims=True)
        pos_ref[:, k:k + 1] = p.astype(jnp.int32)

    offs_col = jnp.sum(jnp.where(ej < ei, cnt_pad_row, 0.0), axis=1,
                       keepdims=True)
    cnt_pad_col = jnp.sum(jnp.where(ei == ej, cnt_pad_row, 0.0), axis=1,
                          keepdims=True)
    ends_col = offs_col + cnt_pad_col
    ti = jax.lax.broadcasted_iota(jnp.int32, (E, 128), 1).astype(jnp.float32)
    cmp = (ti * TM >= ends_col).astype(jnp.int32)
    eid = jnp.minimum(jnp.sum(cmp, axis=0, keepdims=True), E - 1)
    eid_ref[...] = eid.astype(jnp.int32)


def _gate_call(x2, gwt, bias2):
    return pl.pallas_call(
        _gate_body,
        out_shape=[
            jax.ShapeDtypeStruct((T, K), jnp.int32),
            jax.ShapeDtypeStruct((T, E), jnp.float32),
            jax.ShapeDtypeStruct((1, 128), jnp.int32),
        ],
    )(x2, gwt, bias2)


def _silu(x):
    return x / (1.0 + jnp.exp(-x))


def _ffn_body(eid_ref, x_ref, gp_ref, up_ref, dp_ref, o_ref):
    x = x_ref[...]
    h1 = jax.lax.dot_general(x, gp_ref[0], (((1,), (1,)), ((), ())),
                             preferred_element_type=jnp.float32)
    h2 = jax.lax.dot_general(x, up_ref[0], (((1,), (1,)), ((), ())),
                             preferred_element_type=jnp.float32)
    act = _silu(h1) * h2
    o_ref[...] = jax.lax.dot_general(act, dp_ref[0], (((1,), (1,)), ((), ())),
                                     preferred_element_type=jnp.float32)


def _ffn_call(eid, xs, gp, up, dp):
    grid_spec = pltpu.PrefetchScalarGridSpec(
        num_scalar_prefetch=1,
        grid=(NP,),
        in_specs=[
            pl.BlockSpec((TM, H), lambda i, eid: (i, 0)),
            pl.BlockSpec((1, I, H), lambda i, eid: (eid[i], 0, 0)),
            pl.BlockSpec((1, I, H), lambda i, eid: (eid[i], 0, 0)),
            pl.BlockSpec((1, H, I), lambda i, eid: (eid[i], 0, 0)),
        ],
        out_specs=pl.BlockSpec((TM, H), lambda i, eid: (i, 0)),
    )
    return pl.pallas_call(
        _ffn_body,
        grid_spec=grid_spec,
        out_shape=jax.ShapeDtypeStruct((P, H), jnp.float32),
    )(eid, xs, gp, up, dp)


def _shared_body(x_ref, gw_ref, uw_ref, dw_ref, ysg_ref, w_ref, o_ref):
    x = x_ref[...]
    sg = jax.lax.dot_general(x, gw_ref[...], (((1,), (1,)), ((), ())),
                             preferred_element_type=jnp.float32)
    su = jax.lax.dot_general(x, uw_ref[...], (((1,), (1,)), ((), ())),
                             preferred_element_type=jnp.float32)
    act = _silu(sg) * su
    out = jax.lax.dot_general(act, dw_ref[...], (((1,), (1,)), ((), ())),
                              preferred_element_type=jnp.float32)
    for k in range(K):
        out = out + w_ref[:, k:k + 1] * ysg_ref[k]
    o_ref[...] = out


def _shared_call(x2, gw, uw, dw, ysg, w16):
    TS = 256
    return pl.pallas_call(
        _shared_body,
        grid=(T // TS,),
        in_specs=[
            pl.BlockSpec((TS, H), lambda i: (i, 0)),
            pl.BlockSpec((I, H), lambda i: (0, 0)),
            pl.BlockSpec((I, H), lambda i: (0, 0)),
            pl.BlockSpec((H, I), lambda i: (0, 0)),
            pl.BlockSpec((K, TS, H), lambda i: (0, i, 0)),
            pl.BlockSpec((TS, E), lambda i: (i, 0)),
        ],
        out_specs=pl.BlockSpec((TS, H), lambda i: (i, 0)),
        out_shape=jax.ShapeDtypeStruct((T, H), jnp.float32),
    )(x2, gw, uw, dw, ysg, w16)


_SC_MESH = plsc.VectorSubcoreMesh(core_axis_name="c", subcore_axis_name="s")
_CH = 32
_NCH = 256 // _CH


def _dispatch_body(x2_hbm, post_hbm, xs_hbm, xb0, xb1, ib0, ib1, sem0, sem1):
    wid = lax.axis_index("s") * 2 + lax.axis_index("c")
    k = wid // 8
    stripe = wid % 8
    xbufs, ibufs, sems = (xb0, xb1), (ib0, ib1), (sem0, sem1)
    pend = [None, None]
    for c in range(_NCH):
        tb = stripe * 256 + c * _CH
        xb, ib, sm = xbufs[c % 2], ibufs[c % 2], sems[c % 2]
        if pend[c % 2] is not None:
            pend[c % 2].wait()
        pltpu.sync_copy(post_hbm.at[k, pl.ds(tb, _CH)], ib)
        pltpu.sync_copy(x2_hbm.at[pl.ds(tb, _CH)], xb)
        pend[c % 2] = pltpu.async_copy(xb, xs_hbm.at[ib], sm)
    for d in pend:
        if d is not None:
            d.wait()


@functools.partial(
    pl.kernel,
    out_type=jax.ShapeDtypeStruct((P, H), jnp.float32),
    mesh=_SC_MESH,
    scratch_types=[
        pltpu.VMEM((_CH, H), jnp.float32),
        pltpu.VMEM((_CH, H), jnp.float32),
        pltpu.VMEM((_CH,), jnp.int32),
        pltpu.VMEM((_CH,), jnp.int32),
        pltpu.SemaphoreType.DMA,
        pltpu.SemaphoreType.DMA,
    ],
)
def _dispatch_sc(x2, post, xs, xb0, xb1, ib0, ib1, sem0, sem1):
    _dispatch_body(x2, post, xs, xb0, xb1, ib0, ib1, sem0, sem1)


def _gather_body(ys_hbm, post_hbm, ysg_hbm, gb0, gb1, ib0, ib1, sem0, sem1):
    wid = lax.axis_index("s") * 2 + lax.axis_index("c")
    k = wid // 8
    stripe = wid % 8
    gbufs, ibufs, sems = (gb0, gb1), (ib0, ib1), (sem0, sem1)
    pend = [None, None]
    for c in range(_NCH):
        tb = stripe * 256 + c * _CH
        gb, ib, sm = gbufs[c % 2], ibufs[c % 2], sems[c % 2]
        if pend[c % 2] is not None:
            pend[c % 2].wait()
        pltpu.sync_copy(post_hbm.at[k, pl.ds(tb, _CH)], ib)
        pltpu.async_copy(ys_hbm.at[ib], gb, sm).wait()
        pend[c % 2] = pltpu.async_copy(gb, ysg_hbm.at[k, pl.ds(tb, _CH)], sm)
    for d in pend:
        if d is not None:
            d.wait()


@functools.partial(
    pl.kernel,
    out_type=jax.ShapeDtypeStruct((K, T, H), jnp.float32),
    mesh=_SC_MESH,
    scratch_types=[
        pltpu.VMEM((_CH, H), jnp.float32),
        pltpu.VMEM((_CH, H), jnp.float32),
        pltpu.VMEM((_CH,), jnp.int32),
        pltpu.VMEM((_CH,), jnp.int32),
        pltpu.SemaphoreType.DMA,
        pltpu.SemaphoreType.DMA,
    ],
)
def _gather_sc(ys, post, ysg, gb0, gb1, ib0, ib1, sem0, sem1):
    _gather_body(ys, post, ysg, gb0, gb1, ib0, ib1, sem0, sem1)


def kernel(hidden_states, gate_weight, e_score_correction_bias, gate_proj,
           up_proj, down_proj, shared_gate_w, shared_up_w, shared_down_w):
    x2 = hidden_states.reshape(T, H)
    pos, w16, eid_pad = _gate_call(x2, gate_weight.T,
                                   e_score_correction_bias.reshape(1, E))
    eid = eid_pad[0, :NP]
    post = pos.T

    xs = _dispatch_sc(x2, post)
    ys = _ffn_call(eid, xs, gate_proj, up_proj, down_proj)
    ysg = _gather_sc(ys, post)
    out = _shared_call(x2, shared_gate_w, shared_up_w, shared_down_w, ysg, w16)
    return out.reshape(B, S, H)


# speedup vs baseline: 3.8916x; 1.0235x over previous
import functools

import jax
import jax.numpy as jnp
from jax import lax
from jax.experimental import pallas as pl
from jax.experimental.pallas import tpu as pltpu
from jax.experimental.pallas import tpu_sc as plsc

B, S, H = 1, 2048, 1024
I = 512
E = 16
K = 4
G = 4
EPG = E // G
TG = 2
SCALE = 2.5
T = B * S
TM = 256
NP = (T * K) // TM + E
P = NP * TM
H2 = H // 2
NEG = -1e30


def _gate_body(x_ref, gwt_ref, bias_ref, pos_ref, w_ref, eid_ref):
    x = x_ref[...]
    logits = jnp.dot(x, gwt_ref[...], preferred_element_type=jnp.float32)
    scores = 1.0 / (1.0 + jnp.exp(-logits))
    sfc = scores + bias_ref[...]
    lane = jax.lax.broadcasted_iota(jnp.int32, (T, E), 1)

    gs = jnp.zeros((T, G), jnp.float32)
    lane4 = jax.lax.broadcasted_iota(jnp.int32, (T, G), 1)
    for g in range(G):
        m = (lane // EPG) == g
        vals = jnp.where(m, sfc, NEG)
        m1 = jnp.max(vals, axis=-1, keepdims=True)
        idx1 = jnp.min(jnp.where((vals == m1) & m, lane, E), axis=-1,
                       keepdims=True)
        m2 = jnp.max(jnp.where(lane == idx1, NEG, vals), axis=-1,
                     keepdims=True)
        gs = gs + jnp.where(lane4 == g, m1 + m2, 0.0)

    g1v = jnp.max(gs, axis=-1, keepdims=True)
    g1 = jnp.min(jnp.where(gs == g1v, lane4, G), axis=-1, keepdims=True)
    gs2 = jnp.where(lane4 == g1, NEG, gs)
    g2v = jnp.max(gs2, axis=-1, keepdims=True)
    g2 = jnp.min(jnp.where(gs2 == g2v, lane4, G), axis=-1, keepdims=True)
    grp = lane // EPG
    smask = (grp == g1) | (grp == g2)
    tmp = jnp.where(smask, sfc, 0.0)

    oh_k = []
    w_cols = jnp.zeros((T, E), jnp.float32)
    cur = tmp
    for k in range(K):
        mk = jnp.max(cur, axis=-1, keepdims=True)
        ik = jnp.min(jnp.where(cur == mk, lane, E), axis=-1, keepdims=True)
        sel = (lane == ik)
        wk = jnp.sum(jnp.where(sel, scores, 0.0), axis=-1, keepdims=True)
        oh_k.append(sel.astype(jnp.float32))
        w_cols = w_cols + jnp.where(lane == k, wk, 0.0)
        cur = jnp.where(sel, NEG, cur)
    wsum = jnp.sum(jnp.where(lane < K, w_cols, 0.0), axis=-1, keepdims=True)
    w_ref[...] = w_cols / (wsum + 1e-20) * SCALE

    OH = oh_k[0] + oh_k[1] + oh_k[2] + oh_k[3]
    ii = jax.lax.broadcasted_iota(jnp.int32, (T, T), 0)
    jj = jax.lax.broadcasted_iota(jnp.int32, (T, T), 1)
    Lstrict = (jj < ii).astype(jnp.float32)
    CUM = jnp.dot(Lstrict, OH, preferred_element_type=jnp.float32)
    counts_row = jnp.sum(OH, axis=0, keepdims=True)
    cnt_pad_row = jnp.ceil(counts_row / TM) * TM
    ei = jax.lax.broadcasted_iota(jnp.int32, (E, E), 0)
    ej = jax.lax.broadcasted_iota(jnp.int32, (E, E), 1)
    offs_row = jnp.sum(jnp.where(ei < ej, cnt_pad_row.T, 0.0), axis=0,
                       keepdims=True)
    base = offs_row + CUM
    for k in range(K):
        p = jnp.sum(oh_k[k] * base, axis=-1, keepdims=True)
        pos_ref[:, k:k + 1] = p.astype(jnp.int32)

    offs_col = jnp.sum(jnp.where(ej < ei, cnt_pad_row, 0.0), axis=1,
                       keepdims=True)
    cnt_pad_col = jnp.sum(jnp.where(ei == ej, cnt_pad_row, 0.0), axis=1,
                          keepdims=True)
    ends_col = offs_col + cnt_pad_col
    ti = jax.lax.broadcasted_iota(jnp.int32, (E, 128), 1).astype(jnp.float32)
    cmp = (ti * TM >= ends_col).astype(jnp.int32)
    eid = jnp.minimum(jnp.sum(cmp, axis=0, keepdims=True), E - 1)
    eid_ref[...] = eid.astype(jnp.int32)


def _gate_call(x2, gwt, bias2):
    return pl.pallas_call(
        _gate_body,
        out_shape=[
            jax.ShapeDtypeStruct((T, K), jnp.int32),
            jax.ShapeDtypeStruct((T, E), jnp.float32),
            jax.ShapeDtypeStruct((1, 128), jnp.int32),
        ],
    )(x2, gwt, bias2)


def _silu(x):
    return x / (1.0 + jnp.exp(-x))


def _ffn_body(eid_ref, x_ref, gp_ref, up_ref, dp_ref, o_ref):
    x = x_ref[...]
    h1 = jax.lax.dot_general(x, gp_ref[0], (((1,), (1,)), ((), ())),
                             preferred_element_type=jnp.float32)
    h2 = jax.lax.dot_general(x, up_ref[0], (((1,), (1,)), ((), ())),
                             preferred_element_type=jnp.float32)
    act = _silu(h1) * h2
    o_ref[...] = jax.lax.dot_general(act, dp_ref[0], (((1,), (1,)), ((), ())),
                                     preferred_element_type=jnp.float32)


def _ffn_call(eid, xs, gp, up, dp):
    grid_spec = pltpu.PrefetchScalarGridSpec(
        num_scalar_prefetch=1,
        grid=(NP,),
        in_specs=[
            pl.BlockSpec((TM, H), lambda i, eid: (i, 0)),
            pl.BlockSpec((1, I, H), lambda i, eid: (eid[i], 0, 0)),
            pl.BlockSpec((1, I, H), lambda i, eid: (eid[i], 0, 0)),
            pl.BlockSpec((1, H, I), lambda i, eid: (eid[i], 0, 0)),
        ],
        out_specs=pl.BlockSpec((TM, H), lambda i, eid: (i, 0)),
    )
    return pl.pallas_call(
        _ffn_body,
        grid_spec=grid_spec,
        out_shape=jax.ShapeDtypeStruct((P, H), jnp.float32),
    )(eid, xs, gp, up, dp)


def _shared_body(x_ref, gw_ref, uw_ref, dw_ref, ysg_ref, w_ref, o_ref):
    x = x_ref[...]
    sg = jax.lax.dot_general(x, gw_ref[...], (((1,), (1,)), ((), ())),
                             preferred_element_type=jnp.float32)
    su = jax.lax.dot_general(x, uw_ref[...], (((1,), (1,)), ((), ())),
                             preferred_element_type=jnp.float32)
    act = _silu(sg) * su
    out = jax.lax.dot_general(act, dw_ref[...], (((1,), (1,)), ((), ())),
                              preferred_element_type=jnp.float32)
    for k in range(K):
        out = out + w_ref[:, k:k + 1] * ysg_ref[k]
    o_ref[...] = out


def _shared_call(x2, gw, uw, dw, ysg, w16):
    TS = 256
    return pl.pallas_call(
        _shared_body,
        grid=(T // TS,),
        in_specs=[
            pl.BlockSpec((TS, H), lambda i: (i, 0)),
            pl.BlockSpec((I, H), lambda i: (0, 0)),
            pl.BlockSpec((I, H), lambda i: (0, 0)),
            pl.BlockSpec((H, I), lambda i: (0, 0)),
            pl.BlockSpec((K, TS, H), lambda i: (0, i, 0)),
            pl.BlockSpec((TS, E), lambda i: (i, 0)),
        ],
        out_specs=pl.BlockSpec((TS, H), lambda i: (i, 0)),
        out_shape=jax.ShapeDtypeStruct((T, H), jnp.float32),
    )(x2, gw, uw, dw, ysg, w16)


_SC_MESH = plsc.VectorSubcoreMesh(core_axis_name="c", subcore_axis_name="s")
_NW = 32
_CH = 32
_NCH = 256 // _CH


def _dispatch_body(x2_hbm, postr_hbm, xs_hbm, xb0, xb1, iball, sem0, sem1):
    wid = lax.axis_index("s") * 2 + lax.axis_index("c")
    stripe = wid % 8
    xbufs, sems = (xb0, xb1), (sem0, sem1)
    pltpu.sync_copy(postr_hbm.at[wid], iball)
    pend = [None, None]
    for c in range(_NCH):
        tb = stripe * 256 + c * _CH
        xb, sm = xbufs[c % 2], sems[c % 2]
        if pend[c % 2] is not None:
            pend[c % 2].wait()
        pltpu.sync_copy(x2_hbm.at[pl.ds(tb, _CH)], xb)
        pend[c % 2] = pltpu.async_copy(xb, xs_hbm.at[iball.at[c]], sm)
    for d in pend:
        if d is not None:
            d.wait()


@functools.partial(
    pl.kernel,
    out_type=jax.ShapeDtypeStruct((P, H), jnp.float32),
    mesh=_SC_MESH,
    scratch_types=[
        pltpu.VMEM((_CH, H), jnp.float32),
        pltpu.VMEM((_CH, H), jnp.float32),
        pltpu.VMEM((_NCH, _CH), jnp.int32),
        pltpu.SemaphoreType.DMA,
        pltpu.SemaphoreType.DMA,
    ],
)
def _dispatch_sc(x2, postr, xs, xb0, xb1, iball, sem0, sem1):
    _dispatch_body(x2, postr, xs, xb0, xb1, iball, sem0, sem1)


def _gather_body(ys_hbm, postr_hbm, ysg_hbm, gb0, gb1, iball, sem0, sem1,
                 wsem0, wsem1):
    wid = lax.axis_index("s") * 2 + lax.axis_index("c")
    k = wid // 8
    stripe = wid % 8
    gbufs, gsems, wsems = (gb0, gb1), (sem0, sem1), (wsem0, wsem1)
    pltpu.sync_copy(postr_hbm.at[wid], iball)
    pg = [None, None]
    pw = [None, None]
    pg[0] = pltpu.async_copy(ys_hbm.at[iball.at[0]], gb0, sem0)
    pg[1] = pltpu.async_copy(ys_hbm.at[iball.at[1]], gb1, sem1)
    for c in range(_NCH):
        tb = stripe * 256 + c * _CH
        gb = gbufs[c % 2]
        pg[c % 2].wait()
        pw[c % 2] = pltpu.async_copy(gb, ysg_hbm.at[k, pl.ds(tb, _CH)],
                                     wsems[c % 2])
        if c + 2 < _NCH:
            pw[c % 2].wait()
            pg[c % 2] = pltpu.async_copy(ys_hbm.at[iball.at[c + 2]], gb,
                                         gsems[c % 2])
    pw[(_NCH - 2) % 2].wait()
    pw[(_NCH - 1) % 2].wait()


@functools.partial(
    pl.kernel,
    out_type=jax.ShapeDtypeStruct((K, T, H), jnp.float32),
    mesh=_SC_MESH,
    scratch_types=[
        pltpu.VMEM((_CH, H), jnp.float32),
        pltpu.VMEM((_CH, H), jnp.float32),
        pltpu.VMEM((_NCH, _CH), jnp.int32),
        pltpu.SemaphoreType.DMA,
        pltpu.SemaphoreType.DMA,
        pltpu.SemaphoreType.DMA,
        pltpu.SemaphoreType.DMA,
    ],
)
def _gather_sc(ys, postr, ysg, gb0, gb1, iball, sem0, sem1, wsem0, wsem1):
    _gather_body(ys, postr, ysg, gb0, gb1, iball, sem0, sem1, wsem0, wsem1)


def kernel(hidden_states, gate_weight, e_score_correction_bias, gate_proj,
           up_proj, down_proj, shared_gate_w, shared_up_w, shared_down_w):
    x2 = hidden_states.reshape(T, H)
    pos, w16, eid_pad = _gate_call(x2, gate_weight.T,
                                   e_score_correction_bias.reshape(1, E))
    eid = eid_pad[0, :NP]
    post = pos.T

    postr = post.reshape(_NW, _NCH, _CH)
    xs = _dispatch_sc(x2, postr)
    ys = _ffn_call(eid, xs, gate_proj, up_proj, down_proj)
    ysg = _gather_sc(ys, postr)
    out = _shared_call(x2, shared_gate_w, shared_up_w, shared_down_w, ysg, w16)
    return out.reshape(B, S, H)


# speedup vs baseline: 3.9530x; 1.0158x over previous
import functools

import jax
import jax.numpy as jnp
from jax import lax
from jax.experimental import pallas as pl
from jax.experimental.pallas import tpu as pltpu
from jax.experimental.pallas import tpu_sc as plsc

B, S, H = 1, 2048, 1024
I = 512
E = 16
K = 4
G = 4
EPG = E // G
TG = 2
SCALE = 2.5
T = B * S
TM = 256
NP = (T * K) // TM + E
P = NP * TM
H2 = H // 2
NEG = -1e30


def _gate_body(x_ref, gwt_ref, bias_ref, pos_ref, w_ref, eid_ref, vld_ref):
    x = x_ref[...]
    logits = jnp.dot(x, gwt_ref[...], preferred_element_type=jnp.float32)
    scores = 1.0 / (1.0 + jnp.exp(-logits))
    sfc = scores + bias_ref[...]
    lane = jax.lax.broadcasted_iota(jnp.int32, (T, E), 1)

    gs = jnp.zeros((T, G), jnp.float32)
    lane4 = jax.lax.broadcasted_iota(jnp.int32, (T, G), 1)
    for g in range(G):
        m = (lane // EPG) == g
        vals = jnp.where(m, sfc, NEG)
        m1 = jnp.max(vals, axis=-1, keepdims=True)
        idx1 = jnp.min(jnp.where((vals == m1) & m, lane, E), axis=-1,
                       keepdims=True)
        m2 = jnp.max(jnp.where(lane == idx1, NEG, vals), axis=-1,
                     keepdims=True)
        gs = gs + jnp.where(lane4 == g, m1 + m2, 0.0)

    g1v = jnp.max(gs, axis=-1, keepdims=True)
    g1 = jnp.min(jnp.where(gs == g1v, lane4, G), axis=-1, keepdims=True)
    gs2 = jnp.where(lane4 == g1, NEG, gs)
    g2v = jnp.max(gs2, axis=-1, keepdims=True)
    g2 = jnp.min(jnp.where(gs2 == g2v, lane4, G), axis=-1, keepdims=True)
    grp = lane // EPG
    smask = (grp == g1) | (grp == g2)
    tmp = jnp.where(smask, sfc, 0.0)

    oh_k = []
    w_cols = jnp.zeros((T, E), jnp.float32)
    cur = tmp
    for k in range(K):
        mk = jnp.max(cur, axis=-1, keepdims=True)
        ik = jnp.min(jnp.where(cur == mk, lane, E), axis=-1, keepdims=True)
        sel = (lane == ik)
        wk = jnp.sum(jnp.where(sel, scores, 0.0), axis=-1, keepdims=True)
        oh_k.append(sel.astype(jnp.float32))
        w_cols = w_cols + jnp.where(lane == k, wk, 0.0)
        cur = jnp.where(sel, NEG, cur)
    wsum = jnp.sum(jnp.where(lane < K, w_cols, 0.0), axis=-1, keepdims=True)
    w_ref[...] = w_cols / (wsum + 1e-20) * SCALE

    OH = oh_k[0] + oh_k[1] + oh_k[2] + oh_k[3]
    ii = jax.lax.broadcasted_iota(jnp.int32, (T, T), 0)
    jj = jax.lax.broadcasted_iota(jnp.int32, (T, T), 1)
    Lstrict = (jj < ii).astype(jnp.float32)
    CUM = jnp.dot(Lstrict, OH, preferred_element_type=jnp.float32)
    counts_row = jnp.sum(OH, axis=0, keepdims=True)
    cnt_pad_row = jnp.ceil(counts_row / TM) * TM
    ei = jax.lax.broadcasted_iota(jnp.int32, (E, E), 0)
    ej = jax.lax.broadcasted_iota(jnp.int32, (E, E), 1)
    offs_row = jnp.sum(jnp.where(ei < ej, cnt_pad_row.T, 0.0), axis=0,
                       keepdims=True)
    base = offs_row + CUM
    for k in range(K):
        p = jnp.sum(oh_k[k] * base, axis=-1, keepdims=True)
        pos_ref[:, k:k + 1] = p.astype(jnp.int32)

    offs_col = jnp.sum(jnp.where(ej < ei, cnt_pad_row, 0.0), axis=1,
                       keepdims=True)
    cnt_pad_col = jnp.sum(jnp.where(ei == ej, cnt_pad_row, 0.0), axis=1,
                          keepdims=True)
    ends_col = offs_col + cnt_pad_col
    ti = jax.lax.broadcasted_iota(jnp.int32, (E, 128), 1).astype(jnp.float32)
    cmp = (ti * TM >= ends_col).astype(jnp.int32)
    eid = jnp.minimum(jnp.sum(cmp, axis=0, keepdims=True), E - 1)
    used_end_col = offs_col + jnp.sum(jnp.where(ei == ej, counts_row, 0.0),
                                      axis=1, keepdims=True)
    oh2 = (eid == jax.lax.broadcasted_iota(jnp.int32, (E, 128), 0))
    valid = jnp.sum(jnp.where(oh2 & (ti * TM < used_end_col), 1, 0),
                    axis=0, keepdims=True)
    eid_ref[...] = eid.astype(jnp.int32)
    vld_ref[...] = valid.astype(jnp.int32)


def _gate_call(x2, gwt, bias2):
    return pl.pallas_call(
        _gate_body,
        out_shape=[
            jax.ShapeDtypeStruct((T, K), jnp.int32),
            jax.ShapeDtypeStruct((T, E), jnp.float32),
            jax.ShapeDtypeStruct((1, 128), jnp.int32),
            jax.ShapeDtypeStruct((1, 128), jnp.int32),
        ],
    )(x2, gwt, bias2)


def _silu(x):
    return x / (1.0 + jnp.exp(-x))


def _ffn_body(eid_ref, x_ref, gp_ref, up_ref, dp_ref, o_ref):
    i = pl.program_id(0)

    @pl.when(eid_ref[64 + i] > 0)
    def _():
        x = x_ref[...]
        h1 = jax.lax.dot_general(x, gp_ref[0], (((1,), (1,)), ((), ())),
                                 preferred_element_type=jnp.float32)
        h2 = jax.lax.dot_general(x, up_ref[0], (((1,), (1,)), ((), ())),
                                 preferred_element_type=jnp.float32)
        act = _silu(h1) * h2
        o_ref[...] = jax.lax.dot_general(act, dp_ref[0],
                                         (((1,), (1,)), ((), ())),
                                         preferred_element_type=jnp.float32)


def _ffn_call(eid, xs, gp, up, dp):
    grid_spec = pltpu.PrefetchScalarGridSpec(
        num_scalar_prefetch=1,
        grid=(NP,),
        in_specs=[
            pl.BlockSpec((TM, H), lambda i, eid: (i, 0)),
            pl.BlockSpec((1, I, H), lambda i, eid: (eid[i], 0, 0)),
            pl.BlockSpec((1, I, H), lambda i, eid: (eid[i], 0, 0)),
            pl.BlockSpec((1, H, I), lambda i, eid: (eid[i], 0, 0)),
        ],
        out_specs=pl.BlockSpec((TM, H), lambda i, eid: (i, 0)),
    )
    return pl.pallas_call(
        _ffn_body,
        grid_spec=grid_spec,
        out_shape=jax.ShapeDtypeStruct((P, H), jnp.float32),
    )(eid, xs, gp, up, dp)


def _shared_body(x_ref, gw_ref, uw_ref, dw_ref, o_ref):
    x = x_ref[...]
    sg = jax.lax.dot_general(x, gw_ref[...], (((1,), (1,)), ((), ())),
                             preferred_element_type=jnp.float32)
    su = jax.lax.dot_general(x, uw_ref[...], (((1,), (1,)), ((), ())),
                             preferred_element_type=jnp.float32)
    act = _silu(sg) * su
    o_ref[...] = jax.lax.dot_general(act, dw_ref[...], (((1,), (1,)), ((), ())),
                                     preferred_element_type=jnp.float32)


def _combine_body(sh_ref, ysg_ref, w_ref, o_ref):
    out = sh_ref[...]
    for k in range(K):
        out = out + w_ref[:, k:k + 1] * ysg_ref[k]
    o_ref[...] = out


def _combine_call(shared, ysg, w16):
    TS = 512
    return pl.pallas_call(
        _combine_body,
        grid=(T // TS,),
        in_specs=[
            pl.BlockSpec((TS, H), lambda i: (i, 0)),
            pl.BlockSpec((K, TS, H), lambda i: (0, i, 0)),
            pl.BlockSpec((TS, E), lambda i: (i, 0)),
        ],
        out_specs=pl.BlockSpec((TS, H), lambda i: (i, 0)),
        out_shape=jax.ShapeDtypeStruct((T, H), jnp.float32),
    )(shared, ysg, w16)


def _shared_call(x2, gw, uw, dw):
    TS = 256
    return pl.pallas_call(
        _shared_body,
        grid=(T // TS,),
        in_specs=[
            pl.BlockSpec((TS, H), lambda i: (i, 0)),
            pl.BlockSpec((I, H), lambda i: (0, 0)),
            pl.BlockSpec((I, H), lambda i: (0, 0)),
            pl.BlockSpec((H, I), lambda i: (0, 0)),
        ],
        out_specs=pl.BlockSpec((TS, H), lambda i: (i, 0)),
        out_shape=jax.ShapeDtypeStruct((T, H), jnp.float32),
    )(x2, gw, uw, dw)


_SC_MESH = plsc.VectorSubcoreMesh(core_axis_name="c", subcore_axis_name="s")
_NW = 32
_CH = 32
_NCH = 256 // _CH


def _dispatch_body(x2_hbm, postr_hbm, xs_hbm, xb0, xb1, iball, sem0, sem1):
    wid = lax.axis_index("s") * 2 + lax.axis_index("c")
    stripe = wid % 8
    xbufs, sems = (xb0, xb1), (sem0, sem1)
    pltpu.sync_copy(postr_hbm.at[wid], iball)
    pend = [None, None]
    for c in range(_NCH):
        tb = stripe * 256 + c * _CH
        xb, sm = xbufs[c % 2], sems[c % 2]
        if pend[c % 2] is not None:
            pend[c % 2].wait()
        pltpu.sync_copy(x2_hbm.at[pl.ds(tb, _CH)], xb)
        pend[c % 2] = pltpu.async_copy(xb, xs_hbm.at[iball.at[c]], sm)
    for d in pend:
        if d is not None:
            d.wait()


@functools.partial(
    pl.kernel,
    out_type=jax.ShapeDtypeStruct((P, H), jnp.float32),
    mesh=_SC_MESH,
    scratch_types=[
        pltpu.VMEM((_CH, H), jnp.float32),
        pltpu.VMEM((_CH, H), jnp.float32),
        pltpu.VMEM((_NCH, _CH), jnp.int32),
        pltpu.SemaphoreType.DMA,
        pltpu.SemaphoreType.DMA,
    ],
)
def _dispatch_sc(x2, postr, xs, xb0, xb1, iball, sem0, sem1):
    _dispatch_body(x2, postr, xs, xb0, xb1, iball, sem0, sem1)


def _gather_body(ys_hbm, postr_hbm, ysg_hbm, gb0, gb1, iball, sem0, sem1,
                 wsem0, wsem1):
    wid = lax.axis_index("s") * 2 + lax.axis_index("c")
    k = wid // 8
    stripe = wid % 8
    gbufs, gsems, wsems = (gb0, gb1), (sem0, sem1), (wsem0, wsem1)
    pltpu.sync_copy(postr_hbm.at[wid], iball)
    pg = [None, None]
    pw = [None, None]
    pg[0] = pltpu.async_copy(ys_hbm.at[iball.at[0]], gb0, sem0)
    pg[1] = pltpu.async_copy(ys_hbm.at[iball.at[1]], gb1, sem1)
    for c in range(_NCH):
        tb = stripe * 256 + c * _CH
        gb = gbufs[c % 2]
        pg[c % 2].wait()
        pw[c % 2] = pltpu.async_copy(gb, ysg_hbm.at[k, pl.ds(tb, _CH)],
                                     wsems[c % 2])
        if c + 2 < _NCH:
            pw[c % 2].wait()
            pg[c % 2] = pltpu.async_copy(ys_hbm.at[iball.at[c + 2]], gb,
                                         gsems[c % 2])
    pw[(_NCH - 2) % 2].wait()
    pw[(_NCH - 1) % 2].wait()


@functools.partial(
    pl.kernel,
    out_type=jax.ShapeDtypeStruct((K, T, H), jnp.float32),
    mesh=_SC_MESH,
    scratch_types=[
        pltpu.VMEM((_CH, H), jnp.float32),
        pltpu.VMEM((_CH, H), jnp.float32),
        pltpu.VMEM((_NCH, _CH), jnp.int32),
        pltpu.SemaphoreType.DMA,
        pltpu.SemaphoreType.DMA,
        pltpu.SemaphoreType.DMA,
        pltpu.SemaphoreType.DMA,
    ],
)
def _gather_sc(ys, postr, ysg, gb0, gb1, iball, sem0, sem1, wsem0, wsem1):
    _gather_body(ys, postr, ysg, gb0, gb1, iball, sem0, sem1, wsem0, wsem1)


def kernel(hidden_states, gate_weight, e_score_correction_bias, gate_proj,
           up_proj, down_proj, shared_gate_w, shared_up_w, shared_down_w):
    x2 = hidden_states.reshape(T, H)
    pos, w16, eid_pad, vld_pad = _gate_call(x2, gate_weight.T,
                                            e_score_correction_bias.reshape(1, E))
    eid = jnp.concatenate([eid_pad[0, :64], vld_pad[0, :64]])
    post = pos.T

    postr = post.reshape(_NW, _NCH, _CH)
    xs = _dispatch_sc(x2, postr)
    shared = _shared_call(x2, shared_gate_w, shared_up_w, shared_down_w)
    ys = _ffn_call(eid, xs, gate_proj, up_proj, down_proj)
    ysg = _gather_sc(ys, postr)
    out = _combine_call(shared, ysg, w16)
    return out.reshape(B, S, H)


# speedup vs baseline: 3.9838x; 1.0078x over previous
import functools

import jax
import jax.numpy as jnp
from jax import lax
from jax.experimental import pallas as pl
from jax.experimental.pallas import tpu as pltpu
from jax.experimental.pallas import tpu_sc as plsc

B, S, H = 1, 2048, 1024
I = 512
E = 16
K = 4
G = 4
EPG = E // G
TG = 2
SCALE = 2.5
T = B * S
TM = 256
NP = (T * K) // TM + E
P = NP * TM
NEG = -1e30


def _gate_body(x_ref, gwt_ref, bias_ref, pos_ref, w_ref, eid_ref, vld_ref):
    x = x_ref[...]
    logits = jnp.dot(x, gwt_ref[...], preferred_element_type=jnp.float32)
    scores = 1.0 / (1.0 + jnp.exp(-logits))
    sfc = scores + bias_ref[...]
    lane = jax.lax.broadcasted_iota(jnp.int32, (T, E), 1)

    gs = jnp.zeros((T, G), jnp.float32)
    lane4 = jax.lax.broadcasted_iota(jnp.int32, (T, G), 1)
    for g in range(G):
        m = (lane // EPG) == g
        vals = jnp.where(m, sfc, NEG)
        m1 = jnp.max(vals, axis=-1, keepdims=True)
        idx1 = jnp.min(jnp.where((vals == m1) & m, lane, E), axis=-1,
                       keepdims=True)
        m2 = jnp.max(jnp.where(lane == idx1, NEG, vals), axis=-1,
                     keepdims=True)
        gs = gs + jnp.where(lane4 == g, m1 + m2, 0.0)

    g1v = jnp.max(gs, axis=-1, keepdims=True)
    g1 = jnp.min(jnp.where(gs == g1v, lane4, G), axis=-1, keepdims=True)
    gs2 = jnp.where(lane4 == g1, NEG, gs)
    g2v = jnp.max(gs2, axis=-1, keepdims=True)
    g2 = jnp.min(jnp.where(gs2 == g2v, lane4, G), axis=-1, keepdims=True)
    grp = lane // EPG
    smask = (grp == g1) | (grp == g2)
    tmp = jnp.where(smask, sfc, 0.0)

    oh_k = []
    w_cols = jnp.zeros((T, E), jnp.float32)
    cur = tmp
    for k in range(K):
        mk = jnp.max(cur, axis=-1, keepdims=True)
        ik = jnp.min(jnp.where(cur == mk, lane, E), axis=-1, keepdims=True)
        sel = (lane == ik)
        wk = jnp.sum(jnp.where(sel, scores, 0.0), axis=-1, keepdims=True)
        oh_k.append(sel.astype(jnp.float32))
        w_cols = w_cols + jnp.where(lane == k, wk, 0.0)
        cur = jnp.where(sel, NEG, cur)
    wsum = jnp.sum(jnp.where(lane < K, w_cols, 0.0), axis=-1, keepdims=True)
    w_ref[...] = w_cols / (wsum + 1e-20) * SCALE

    OH = oh_k[0] + oh_k[1] + oh_k[2] + oh_k[3]
    ii = jax.lax.broadcasted_iota(jnp.int32, (T, T), 0)
    jj = jax.lax.broadcasted_iota(jnp.int32, (T, T), 1)
    Lstrict = (jj < ii).astype(jnp.float32)
    CUM = jnp.dot(Lstrict, OH, preferred_element_type=jnp.float32)
    counts_row = jnp.sum(OH, axis=0, keepdims=True)
    cnt_pad_row = jnp.ceil(counts_row / TM) * TM
    ei = jax.lax.broadcasted_iota(jnp.int32, (E, E), 0)
    ej = jax.lax.broadcasted_iota(jnp.int32, (E, E), 1)
    offs_row = jnp.sum(jnp.where(ei < ej, cnt_pad_row.T, 0.0), axis=0,
                       keepdims=True)
    base = offs_row + CUM
    for k in range(K):
        p = jnp.sum(oh_k[k] * base, axis=-1, keepdims=True)
        pos_ref[:, k:k + 1] = p.astype(jnp.int32)

    offs_col = jnp.sum(jnp.where(ej < ei, cnt_pad_row, 0.0), axis=1,
                       keepdims=True)
    cnt_pad_col = jnp.sum(jnp.where(ei == ej, cnt_pad_row, 0.0), axis=1,
                          keepdims=True)
    ends_col = offs_col + cnt_pad_col
    ti = jax.lax.broadcasted_iota(jnp.int32, (E, 128), 1).astype(jnp.float32)
    cmp = (ti * TM >= ends_col).astype(jnp.int32)
    eid = jnp.minimum(jnp.sum(cmp, axis=0, keepdims=True), E - 1)
    used_end_col = offs_col + jnp.sum(jnp.where(ei == ej, counts_row, 0.0),
                                      axis=1, keepdims=True)
    oh2 = (eid == jax.lax.broadcasted_iota(jnp.int32, (E, 128), 0))
    valid = jnp.sum(jnp.where(oh2 & (ti * TM < used_end_col), 1, 0),
                    axis=0, keepdims=True)
    eid_ref[...] = eid.astype(jnp.int32)
    vld_ref[...] = valid.astype(jnp.int32)


def _gate_call(x2, gwt, bias2):
    return pl.pallas_call(
        _gate_body,
        out_shape=[
            jax.ShapeDtypeStruct((T, K), jnp.int32),
            jax.ShapeDtypeStruct((T, E), jnp.float32),
            jax.ShapeDtypeStruct((1, 128), jnp.int32),
            jax.ShapeDtypeStruct((1, 128), jnp.int32),
        ],
    )(x2, gwt, bias2)


def _silu(x):
    return x / (1.0 + jnp.exp(-x))


def _ffn_body(eid_ref, x_ref, gp_ref, up_ref, dp_ref, o_ref):
    i = pl.program_id(0)

    @pl.when(eid_ref[64 + i] > 0)
    def _():
        x = x_ref[...]
        h1 = jax.lax.dot_general(x, gp_ref[0], (((1,), (1,)), ((), ())),
                                 preferred_element_type=jnp.float32)
        h2 = jax.lax.dot_general(x, up_ref[0], (((1,), (1,)), ((), ())),
                                 preferred_element_type=jnp.float32)
        act = _silu(h1) * h2
        o_ref[...] = jax.lax.dot_general(act, dp_ref[0],
                                         (((1,), (1,)), ((), ())),
                                         preferred_element_type=jnp.float32)


def _ffn_call(eid, xs, gp, up, dp):
    grid_spec = pltpu.PrefetchScalarGridSpec(
        num_scalar_prefetch=1,
        grid=(NP,),
        in_specs=[
            pl.BlockSpec((TM, H), lambda i, eid: (i, 0)),
            pl.BlockSpec((1, I, H), lambda i, eid: (eid[i], 0, 0)),
            pl.BlockSpec((1, I, H), lambda i, eid: (eid[i], 0, 0)),
            pl.BlockSpec((1, H, I), lambda i, eid: (eid[i], 0, 0)),
        ],
        out_specs=pl.BlockSpec((TM, H), lambda i, eid: (i, 0)),
    )
    return pl.pallas_call(
        _ffn_body,
        grid_spec=grid_spec,
        out_shape=jax.ShapeDtypeStruct((P, H), jnp.float32),
    )(eid, xs, gp, up, dp)


def _shared_body(x_ref, gw_ref, uw_ref, dw_ref, o_ref):
    x = x_ref[...]
    sg = jax.lax.dot_general(x, gw_ref[...], (((1,), (1,)), ((), ())),
                             preferred_element_type=jnp.float32)
    su = jax.lax.dot_general(x, uw_ref[...], (((1,), (1,)), ((), ())),
                             preferred_element_type=jnp.float32)
    act = _silu(sg) * su
    o_ref[...] = jax.lax.dot_general(act, dw_ref[...], (((1,), (1,)), ((), ())),
                                     preferred_element_type=jnp.float32)


def _combine_body(sh_ref, ysg_ref, w_ref, o_ref):
    out = sh_ref[...]
    for k in range(K):
        out = out + w_ref[:, k:k + 1] * ysg_ref[k]
    o_ref[...] = out


def _combine_call(shared, ysg, w16):
    TS = 512
    return pl.pallas_call(
        _combine_body,
        grid=(T // TS,),
        in_specs=[
            pl.BlockSpec((TS, H), lambda i: (i, 0)),
            pl.BlockSpec((K, TS, H), lambda i: (0, i, 0)),
            pl.BlockSpec((TS, E), lambda i: (i, 0)),
        ],
        out_specs=pl.BlockSpec((TS, H), lambda i: (i, 0)),
        out_shape=jax.ShapeDtypeStruct((T, H), jnp.float32),
    )(shared, ysg, w16)


def _shared_call(x2, gw, uw, dw):
    TS = 256
    return pl.pallas_call(
        _shared_body,
        grid=(T // TS,),
        in_specs=[
            pl.BlockSpec((TS, H), lambda i: (i, 0)),
            pl.BlockSpec((I, H), lambda i: (0, 0)),
            pl.BlockSpec((I, H), lambda i: (0, 0)),
            pl.BlockSpec((H, I), lambda i: (0, 0)),
        ],
        out_specs=pl.BlockSpec((TS, H), lambda i: (i, 0)),
        out_shape=jax.ShapeDtypeStruct((T, H), jnp.float32),
    )(x2, gw, uw, dw)


_SC_MESH = plsc.VectorSubcoreMesh(core_axis_name="c", subcore_axis_name="s")
_NW = 32
_CH = 32
_NCH = 256 // _CH


def _dispatch_body(x2_hbm, postr_hbm, xs_hbm, xb0, xb1, iball, sem0, sem1):
    wid = lax.axis_index("s") * 2 + lax.axis_index("c")
    stripe = wid % 8
    xbufs, sems = (xb0, xb1), (sem0, sem1)
    pltpu.sync_copy(postr_hbm.at[wid], iball)
    pend = [None, None]
    for c in range(_NCH):
        tb = stripe * 256 + c * _CH
        xb, sm = xbufs[c % 2], sems[c % 2]
        if pend[c % 2] is not None:
            pend[c % 2].wait()
        pltpu.sync_copy(x2_hbm.at[pl.ds(tb, _CH)], xb)
        pend[c % 2] = pltpu.async_copy(xb, xs_hbm.at[iball.at[c]], sm)
    for d in pend:
        if d is not None:
            d.wait()


@functools.partial(
    pl.kernel,
    out_type=jax.ShapeDtypeStruct((P, H), jnp.float32),
    mesh=_SC_MESH,
    scratch_types=[
        pltpu.VMEM((_CH, H), jnp.float32),
        pltpu.VMEM((_CH, H), jnp.float32),
        pltpu.VMEM((_NCH, _CH), jnp.int32),
        pltpu.SemaphoreType.DMA,
        pltpu.SemaphoreType.DMA,
    ],
)
def _dispatch_sc(x2, postr, xs, xb0, xb1, iball, sem0, sem1):
    _dispatch_body(x2, postr, xs, xb0, xb1, iball, sem0, sem1)


def _gather_body(ys_hbm, postr_hbm, ysg_hbm, gb0, gb1, iball, sem0, sem1,
                 wsem0, wsem1):
    wid = lax.axis_index("s") * 2 + lax.axis_index("c")
    k = wid // 8
    stripe = wid % 8
    gbufs, gsems, wsems = (gb0, gb1), (sem0, sem1), (wsem0, wsem1)
    pltpu.sync_copy(postr_hbm.at[wid], iball)
    pg = [None, None]
    pw = [None, None]
    pg[0] = pltpu.async_copy(ys_hbm.at[iball.at[0]], gb0, sem0)
    pg[1] = pltpu.async_copy(ys_hbm.at[iball.at[1]], gb1, sem1)
    for c in range(_NCH):
        tb = stripe * 256 + c * _CH
        gb = gbufs[c % 2]
        pg[c % 2].wait()
        pw[c % 2] = pltpu.async_copy(gb, ysg_hbm.at[k, pl.ds(tb, _CH)],
                                     wsems[c % 2])
        if c + 2 < _NCH:
            pw[c % 2].wait()
            pg[c % 2] = pltpu.async_copy(ys_hbm.at[iball.at[c + 2]], gb,
                                         gsems[c % 2])
    pw[(_NCH - 2) % 2].wait()
    pw[(_NCH - 1) % 2].wait()


@functools.partial(
    pl.kernel,
    out_type=jax.ShapeDtypeStruct((K, T, H), jnp.float32),
    mesh=_SC_MESH,
    scratch_types=[
        pltpu.VMEM((_CH, H), jnp.float32),
        pltpu.VMEM((_CH, H), jnp.float32),
        pltpu.VMEM((_NCH, _CH), jnp.int32),
        pltpu.SemaphoreType.DMA,
        pltpu.SemaphoreType.DMA,
        pltpu.SemaphoreType.DMA,
        pltpu.SemaphoreType.DMA,
    ],
)
def _gather_sc(ys, postr, ysg, gb0, gb1, iball, sem0, sem1, wsem0, wsem1):
    _gather_body(ys, postr, ysg, gb0, gb1, iball, sem0, sem1, wsem0, wsem1)


def kernel(hidden_states, gate_weight, e_score_correction_bias, gate_proj,
           up_proj, down_proj, shared_gate_w, shared_up_w, shared_down_w):
    x2 = hidden_states.reshape(T, H)
    pos, w16, eid_pad, vld_pad = _gate_call(x2, gate_weight.T,
                                            e_score_correction_bias.reshape(1, E))
    eid = jnp.concatenate([eid_pad[0, :64], vld_pad[0, :64]])
    post = pos.T

    postr = post.reshape(_NW, _NCH, _CH)
    xs = _dispatch_sc(x2, postr)
    shared = _shared_call(x2, shared_gate_w, shared_up_w, shared_down_w)
    ys = _ffn_call(eid, xs, gate_proj, up_proj, down_proj)
    ysg = _gather_sc(ys, postr)
    out = _combine_call(shared, ysg, w16)
    return out.reshape(B, S, H)
